# baseline (speedup 1.0000x reference)
import functools

import jax
import jax.numpy as jnp
import numpy as np
from jax.experimental import pallas as pl
from jax.experimental.pallas import tpu as pltpu

N_BINS = 15


def _ece_stats_kernel(logits_ref, labels_ref, stats_ref):
    i = pl.program_id(0)
    x = logits_ref[...]
    m = jnp.max(x, axis=1)
    s = jnp.sum(jnp.exp(x), axis=1)
    conf = jnp.exp(m) / s
    pred = jnp.argmax(x, axis=1).astype(jnp.int32)
    acc = (pred == labels_ref[...]).astype(jnp.float32)

    step = jnp.float32(1.0) / jnp.float32(N_BINS)
    bcol = jax.lax.broadcasted_iota(jnp.int32, (N_BINS, 1), 0)
    lowers = bcol.astype(jnp.float32) * step
    uppers = (bcol + 1).astype(jnp.float32) * step

    conf_t = conf[None, :]
    acc_t = acc[None, :]
    in_bin = ((conf_t > lowers)
              & (conf_t <= uppers)).astype(jnp.float32)
    cnt = jnp.sum(in_bin, axis=1, keepdims=True)
    sum_conf = jnp.sum(in_bin * conf_t, axis=1, keepdims=True)
    sum_acc = jnp.sum(in_bin * acc_t, axis=1, keepdims=True)

    @pl.when(i == 0)
    def _init():
        stats_ref[...] = jnp.zeros_like(stats_ref)

    stats_ref[0:N_BINS, 0:1] += cnt
    stats_ref[16:16 + N_BINS, 0:1] += sum_conf
    stats_ref[32:32 + N_BINS, 0:1] += sum_acc


def kernel(logits, labels):
    n_rows, n_cols = logits.shape
    block = 4096
    grid = n_rows // block

    stats48 = pl.pallas_call(
        _ece_stats_kernel,
        grid=(grid,),
        in_specs=[
            pl.BlockSpec((block, n_cols), lambda i: (i, 0)),
            pl.BlockSpec((block,), lambda i: (i,)),
        ],
        out_specs=pl.BlockSpec((48, 128), lambda i: (0, 0)),
        out_shape=jax.ShapeDtypeStruct((48, 128), jnp.float32),
        compiler_params=pltpu.CompilerParams(
            dimension_semantics=("arbitrary",),
        ),
    )(logits, labels)

    cnt = stats48[0:N_BINS, 0]
    sum_conf = stats48[16:16 + N_BINS, 0]
    sum_acc = stats48[32:32 + N_BINS, 0]

    n = jnp.float32(n_rows)
    prop = cnt / n
    safe = jnp.where(cnt > 0, cnt, 1.0)
    avg_conf = sum_conf / safe
    avg_acc = sum_acc / safe
    gaps = jnp.abs(avg_conf - avg_acc) * prop
    ece = jnp.where(cnt > 0, gaps, 0.0).sum().reshape(1)
    prob_out = jnp.where(cnt > 0, avg_conf, 0.0)
    accu_out = jnp.where(cnt > 0, avg_acc, 0.0)
    return (ece, prob_out, accu_out)
